# baseline (speedup 1.0000x reference)
import jax
import jax.numpy as jnp
from jax.experimental import pallas as pl

N = 10000
D_IN = 128
D_OUT = 128
BM = 400


def _gcn_kernel(x_ref, w_ref, b_ref, adj_ref, out_ref):
    tmp = jnp.dot(adj_ref[...], x_ref[...], preferred_element_type=jnp.float32)
    out_ref[...] = (
        jnp.dot(tmp, w_ref[...], preferred_element_type=jnp.float32) + b_ref[...]
    )


@jax.jit
def kernel(x, adj, W, b):
    grid = (N // BM,)
    return pl.pallas_call(
        _gcn_kernel,
        grid=grid,
        in_specs=[
            pl.BlockSpec((N, D_IN), lambda i: (0, 0)),
            pl.BlockSpec((D_IN, D_OUT), lambda i: (0, 0)),
            pl.BlockSpec((1, D_OUT), lambda i: (0, 0)),
            pl.BlockSpec((BM, N), lambda i: (i, 0)),
        ],
        out_specs=pl.BlockSpec((BM, D_OUT), lambda i: (i, 0)),
        out_shape=jax.ShapeDtypeStruct((N, D_OUT), jnp.float32),
    )(x, W, b.reshape(1, D_OUT), adj)
